# baseline (speedup 1.0000x reference)
import functools

import jax
import jax.numpy as jnp
import numpy as np
from jax import lax
from jax.experimental import pallas as pl
from jax.experimental.pallas import tpu as pltpu
from jax.experimental.pallas import tpu_sc as plsc

R = 128
N = 32768
L = 16
NV = N // L
S = 8
VS = NV // S
ES = N // S
POS_SHIFT = 12
HIST = 2048
HV = HIST // L
SIGN = np.int32(-2147483648)

B1_SHIFT, B1_MASK = 11, np.int32(2047)
B2_SHIFT, B2_MASK = 22, np.int32(1023)
D0_MASK = np.int32(2047)

S2 = 16
ES2 = N // S2
VS2 = NV // S2
POS2_SHIFT = 11
HIST2 = 1024


def _zero_hist(hist):
    zeros = jnp.zeros((L,), jnp.int32)

    def body(t, c):
        for u in range(8):
            hist[pl.ds((t * 8 + u) * L, L)] = zeros
        return c

    lax.fori_loop(0, (S * HIST) // (8 * L), body, 0)


def _prefix(hist, offs, ns=S, nbins=HIST):
    zeros = jnp.zeros((L,), jnp.int32)

    def body(t, carry):
        h = [hist[pl.ds(j * nbins + t * L, L)] for j in range(ns)]
        total = h[0]
        for j in range(1, ns):
            total = total + h[j]
        cum = plsc.cumsum(total)
        run = cum - total + carry
        for j in range(ns):
            offs[pl.ds(j * nbins + t * L, L)] = run
            run = run + h[j]
            hist[pl.ds(j * nbins + t * L, L)] = zeros
        return carry + lax.reduce_sum(total, axes=(0,))

    lax.fori_loop(0, nbins // L, body, jnp.int32(-1))


def _sweep0(src, hist):
    ones = jnp.ones((L,), jnp.int32)

    def chain(j, i):
        off = j * ES + i * L
        u = src[pl.ds(off, L)]
        m = lax.shift_right_arithmetic(u, np.int32(31))
        k = lax.bitwise_xor(u, lax.bitwise_or(m, SIGN))
        d = lax.bitwise_and(k, D0_MASK)
        plsc.addupdate_scatter(hist.at[pl.ds(j * HIST, HIST)], [d], ones)

    def body(i, c):
        @plsc.parallel_loop(0, S, 1, unroll=S)
        def _(j):
            chain(j, 2 * i)
            chain(j, 2 * i + 1)
        return c

    lax.fori_loop(0, VS // 2, body, 0)


def _permute(src, dst, offs, hist, shift, mask, next_shift, next_mask,
             next_pos_shift, next_bin_shift, from_raw=False):
    ones = jnp.ones((L,), jnp.int32)

    def front(j, i):
        k = src[pl.ds(j * ES + i * L, L)]
        if from_raw:
            m = lax.shift_right_arithmetic(k, np.int32(31))
            k = lax.bitwise_xor(k, lax.bitwise_or(m, SIGN))
        if shift:
            d = lax.bitwise_and(
                lax.shift_right_logical(k, np.int32(shift)), mask)
        else:
            d = lax.bitwise_and(k, mask)
        cnt, last = plsc.scan_count(d)
        return k, d, cnt, last

    def back(j, k, d, cnt, last):
        offs_j = offs.at[pl.ds(j * HIST, HIST)]
        base = plsc.load_gather(offs_j, [d])
        pos = base + cnt
        plsc.store_scatter(dst, [pos], k)
        plsc.addupdate_scatter(offs_j, [d], cnt, mask=last)
        if next_shift is not None:
            nd = lax.bitwise_and(
                lax.shift_right_logical(k, np.int32(next_shift)),
                next_mask)
            idx = nd + lax.shift_left(
                lax.shift_right_logical(pos, np.int32(next_pos_shift)),
                np.int32(next_bin_shift))
            plsc.addupdate_scatter(hist, [idx], ones)

    def body(i, c):
        @plsc.parallel_loop(0, S, 1, unroll=S)
        def _(j):
            a = front(j, 2 * i)
            b = front(j, 2 * i + 1)
            back(j, *a)
            back(j, *b)
        return c

    lax.fori_loop(0, VS // 2, body, 0)


def _permute_final(src, dst, offs, shift, mask):
    def chain(j, i):
        k = src[pl.ds(j * ES2 + i * L, L)]
        d = lax.bitwise_and(
            lax.shift_right_logical(k, np.int32(shift)), mask)
        offs_j = offs.at[pl.ds(j * HIST2, HIST2)]
        cnt, last = plsc.scan_count(d)
        base = plsc.load_gather(offs_j, [d])
        pos = base + cnt
        t = lax.shift_right_arithmetic(k, np.int32(31))
        u = lax.bitwise_xor(
            k, lax.bitwise_or(lax.bitwise_not(t), SIGN))
        plsc.store_scatter(dst, [pos], u)
        plsc.addupdate_scatter(offs_j, [d], cnt, mask=last)

    def body(i, c):
        @plsc.parallel_loop(0, S2, 1, unroll=S2)
        def _(j):
            chain(j, i)
        return c

    lax.fori_loop(0, VS2, body, 0)


def _sort_body(nc, nw, x_hbm, out_hbm, buf0, buf1, hist, offs,
               sem_in, sem_out):
    wid = lax.axis_index("s") * nc + lax.axis_index("c")
    rows_per = R // nw
    row0 = wid * rows_per
    pltpu.async_copy(x_hbm.at[row0], buf0, sem_in)
    _zero_hist(hist)

    def row_body(r, c):
        row = row0 + r
        pltpu.make_async_copy(x_hbm.at[row], buf0, sem_in).wait()

        _sweep0(buf0, hist)
        _prefix(hist, offs)

        @pl.when(r > 0)
        def _():
            pltpu.make_async_copy(buf1, out_hbm.at[row - 1], sem_out).wait()

        _permute(buf0, buf1, offs, hist, 0, D0_MASK, B1_SHIFT, B1_MASK,
                 POS_SHIFT, 11, from_raw=True)
        _prefix(hist, offs)
        _permute(buf1, buf0, offs, hist, B1_SHIFT, B1_MASK,
                 B2_SHIFT, B2_MASK, POS2_SHIFT, 10)
        _prefix(hist, offs, ns=S2, nbins=HIST2)
        _permute_final(buf0, buf1, offs, B2_SHIFT, B2_MASK)

        pltpu.async_copy(buf1, out_hbm.at[row], sem_out)

        @pl.when(r < rows_per - 1)
        def _():
            pltpu.async_copy(x_hbm.at[row + 1], buf0, sem_in)

        return c

    lax.fori_loop(0, rows_per, row_body, 0)
    last = row0 + rows_per - 1
    pltpu.make_async_copy(buf1, out_hbm.at[last], sem_out).wait()


@jax.jit
def kernel(x):
    try:
        info = plsc.get_sparse_core_info()
        nc, ns = info.num_cores, info.num_subcores
    except Exception:
        nc, ns = 2, 16
    nw = nc * ns
    mesh = plsc.VectorSubcoreMesh(core_axis_name="c", subcore_axis_name="s")
    f = pl.kernel(
        functools.partial(_sort_body, nc, nw),
        out_type=jax.ShapeDtypeStruct((R, N), jnp.int32),
        mesh=mesh,
        compiler_params=pltpu.CompilerParams(needs_layout_passes=False),
        scratch_types=[
            pltpu.VMEM((N,), jnp.int32),
            pltpu.VMEM((N,), jnp.int32),
            pltpu.VMEM((S * HIST,), jnp.int32),
            pltpu.VMEM((S * HIST,), jnp.int32),
            pltpu.SemaphoreType.DMA,
            pltpu.SemaphoreType.DMA,
        ],
    )
    xi = lax.bitcast_convert_type(x, jnp.int32)
    return lax.bitcast_convert_type(f(xi), jnp.float32)


# speedup vs baseline: 1.0364x; 1.0364x over previous
import functools

import jax
import jax.numpy as jnp
import numpy as np
from jax import lax
from jax.experimental import pallas as pl
from jax.experimental.pallas import tpu as pltpu
from jax.experimental.pallas import tpu_sc as plsc

R = 128
N = 32768
L = 16
NV = N // L
S = 8
VS = NV // S
ES = N // S
POS_SHIFT = 12
HIST = 2048
HV = HIST // L
SIGN = np.int32(-2147483648)

B1_SHIFT, B1_MASK = 11, np.int32(2047)
B2_SHIFT, B2_MASK = 22, np.int32(1023)
D0_MASK = np.int32(2047)

S2 = 16
ES2 = N // S2
VS2 = NV // S2
POS2_SHIFT = 11
HIST2 = 1024


def _zero_hist(hist):
    zeros = jnp.zeros((L,), jnp.int32)

    def body(t, c):
        for u in range(8):
            hist[pl.ds((t * 8 + u) * L, L)] = zeros
        return c

    lax.fori_loop(0, (S * HIST) // (8 * L), body, 0)


def _prefix(hist, offs, ns=S, nbins=HIST):
    zeros = jnp.zeros((L,), jnp.int32)

    def body(t, carry):
        h = [hist[pl.ds(j * nbins + t * L, L)] for j in range(ns)]
        total = h[0]
        for j in range(1, ns):
            total = total + h[j]
        cum = plsc.cumsum(total)
        run = cum - total + carry
        for j in range(ns):
            offs[pl.ds(j * nbins + t * L, L)] = run
            run = run + h[j]
            hist[pl.ds(j * nbins + t * L, L)] = zeros
        return carry + lax.reduce_sum(total, axes=(0,))

    lax.fori_loop(0, nbins // L, body, jnp.int32(-1))


def _sweep0(src, hist):
    ones = jnp.ones((L,), jnp.int32)

    def chain(j, i):
        off = j * ES + i * L
        u = src[pl.ds(off, L)]
        m = lax.shift_right_arithmetic(u, np.int32(31))
        k = lax.bitwise_xor(u, lax.bitwise_or(m, SIGN))
        d = lax.bitwise_and(k, D0_MASK)
        plsc.addupdate_scatter(hist.at[pl.ds(j * HIST, HIST)], [d], ones)

    def body(i, c):
        @plsc.parallel_loop(0, S, 1, unroll=S)
        def _(j):
            chain(j, 2 * i)
            chain(j, 2 * i + 1)
        return c

    lax.fori_loop(0, VS // 2, body, 0)


def _permute(src, dst, offs, hist, shift, mask, next_shift, next_mask,
             next_pos_shift, next_bin_shift, from_raw=False):
    ones = jnp.ones((L,), jnp.int32)

    def chain(j, i):
        k = src[pl.ds(j * ES + i * L, L)]
        if from_raw:
            m = lax.shift_right_arithmetic(k, np.int32(31))
            k = lax.bitwise_xor(k, lax.bitwise_or(m, SIGN))
        if shift:
            d = lax.bitwise_and(
                lax.shift_right_logical(k, np.int32(shift)), mask)
        else:
            d = lax.bitwise_and(k, mask)
        offs_j = offs.at[pl.ds(j * HIST, HIST)]
        cnt, last = plsc.scan_count(d)
        base = plsc.load_gather(offs_j, [d])
        pos = base + cnt
        plsc.store_scatter(dst, [pos], k)
        plsc.addupdate_scatter(offs_j, [d], cnt, mask=last)
        if next_shift is not None:
            nd = lax.bitwise_and(
                lax.shift_right_logical(k, np.int32(next_shift)),
                next_mask)
            idx = nd + lax.shift_left(
                lax.shift_right_logical(pos, np.int32(next_pos_shift)),
                np.int32(next_bin_shift))
            plsc.addupdate_scatter(hist, [idx], ones)

    def body(i, c):
        @plsc.parallel_loop(0, S, 1, unroll=S)
        def _(j):
            chain(j, 2 * i)
            chain(j, 2 * i + 1)
        return c

    lax.fori_loop(0, VS // 2, body, 0)


def _permute_final(src, dst, offs, shift, mask):
    def chain(j, i):
        k = src[pl.ds(j * ES2 + i * L, L)]
        d = lax.bitwise_and(
            lax.shift_right_logical(k, np.int32(shift)), mask)
        offs_j = offs.at[pl.ds(j * HIST2, HIST2)]
        cnt, last = plsc.scan_count(d)
        base = plsc.load_gather(offs_j, [d])
        pos = base + cnt
        t = lax.shift_right_arithmetic(k, np.int32(31))
        u = lax.bitwise_xor(
            k, lax.bitwise_or(lax.bitwise_not(t), SIGN))
        plsc.store_scatter(dst, [pos], u)
        plsc.addupdate_scatter(offs_j, [d], cnt, mask=last)

    def body(i, c):
        @plsc.parallel_loop(0, S2, 1, unroll=S2)
        def _(j):
            chain(j, i)
        return c

    lax.fori_loop(0, VS2, body, 0)


def _sort_body(nc, nw, x_hbm, out_hbm, buf0, buf1, hist, offs,
               sem_in, sem_out):
    wid = lax.axis_index("s") * nc + lax.axis_index("c")
    rows_per = R // nw
    row0 = wid * rows_per
    pltpu.async_copy(x_hbm.at[row0], buf0, sem_in)
    _zero_hist(hist)

    def row_body(r, c):
        row = row0 + r
        pltpu.make_async_copy(x_hbm.at[row], buf0, sem_in).wait()

        _sweep0(buf0, hist)
        _prefix(hist, offs)

        @pl.when(r > 0)
        def _():
            pltpu.make_async_copy(buf1, out_hbm.at[row - 1], sem_out).wait()

        _permute(buf0, buf1, offs, hist, 0, D0_MASK, B1_SHIFT, B1_MASK,
                 POS_SHIFT, 11, from_raw=True)
        _prefix(hist, offs)
        _permute(buf1, buf0, offs, hist, B1_SHIFT, B1_MASK,
                 B2_SHIFT, B2_MASK, POS2_SHIFT, 10)
        _prefix(hist, offs, ns=S2, nbins=HIST2)
        _permute_final(buf0, buf1, offs, B2_SHIFT, B2_MASK)

        pltpu.async_copy(buf1, out_hbm.at[row], sem_out)

        @pl.when(r < rows_per - 1)
        def _():
            pltpu.async_copy(x_hbm.at[row + 1], buf0, sem_in)

        return c

    lax.fori_loop(0, rows_per, row_body, 0)
    last = row0 + rows_per - 1
    pltpu.make_async_copy(buf1, out_hbm.at[last], sem_out).wait()


@jax.jit
def kernel(x):
    try:
        info = plsc.get_sparse_core_info()
        nc, ns = info.num_cores, info.num_subcores
    except Exception:
        nc, ns = 2, 16
    nw = nc * ns
    mesh = plsc.VectorSubcoreMesh(core_axis_name="c", subcore_axis_name="s")
    f = pl.kernel(
        functools.partial(_sort_body, nc, nw),
        out_type=jax.ShapeDtypeStruct((R, N), jnp.int32),
        mesh=mesh,
        compiler_params=pltpu.CompilerParams(needs_layout_passes=False),
        scratch_types=[
            pltpu.VMEM((N,), jnp.int32),
            pltpu.VMEM((N,), jnp.int32),
            pltpu.VMEM((S * HIST,), jnp.int32),
            pltpu.VMEM((S * HIST,), jnp.int32),
            pltpu.SemaphoreType.DMA,
            pltpu.SemaphoreType.DMA,
        ],
    )
    xi = lax.bitcast_convert_type(x, jnp.int32)
    return lax.bitcast_convert_type(f(xi), jnp.float32)
